# Initial kernel scaffold; baseline (speedup 1.0000x reference)
import jax
import jax.numpy as jnp
from jax.experimental import pallas as pl


def kernel(x, batch, W1, b1, W2, b2, W3, b3):
    raise NotImplementedError("write your pallas kernel here")


# speedup vs baseline: 3.0589x; 3.0589x over previous
import functools

import jax
import jax.numpy as jnp
from jax.experimental import pallas as pl
from jax.experimental.pallas import tpu as pltpu

N = 100000
D = 128
H = 256
O = 10
G = 256
B = 400
NBLK = N // B


def _seg_mlp_kernel(batch_ref, x_ref, w1_ref, b1_ref, w2_ref, b2_ref,
                    w3_ref, b3_ref, out_ref, acc_ref, cnt_ref):
  i = pl.program_id(0)
  nb = pl.num_programs(0)

  @pl.when(i == 0)
  def _init():
    acc_ref[...] = jnp.zeros_like(acc_ref)
    cnt_ref[...] = jnp.zeros_like(cnt_ref)

  b = batch_ref[0]
  oh = (jax.lax.broadcasted_iota(jnp.int32, (G, B), 0) == b).astype(jnp.float32)
  acc_ref[...] += jnp.dot(oh, x_ref[...], preferred_element_type=jnp.float32)
  cnt_ref[...] += jnp.sum(oh, axis=1, keepdims=True)

  @pl.when(i == nb - 1)
  def _finish():
    pooled = acc_ref[...] / jnp.maximum(cnt_ref[...], 1.0)
    h = jnp.maximum(
        jnp.dot(pooled, w1_ref[...], preferred_element_type=jnp.float32)
        + b1_ref[...], 0.0)
    h = jnp.maximum(
        jnp.dot(h, w2_ref[...], preferred_element_type=jnp.float32)
        + b2_ref[...], 0.0)
    out_ref[...] = (
        jnp.dot(h, w3_ref[...], preferred_element_type=jnp.float32)
        + b3_ref[...])


@jax.jit
def kernel(x, batch, W1, b1, W2, b2, W3, b3):
  batch3d = batch.astype(jnp.int32).reshape(NBLK, 1, B)
  out = pl.pallas_call(
      _seg_mlp_kernel,
      grid=(NBLK,),
      in_specs=[
          pl.BlockSpec((1, 1, B), lambda i: (i, 0, 0)),
          pl.BlockSpec((B, D), lambda i: (i, 0)),
          pl.BlockSpec((D, H), lambda i: (0, 0)),
          pl.BlockSpec((1, H), lambda i: (0, 0)),
          pl.BlockSpec((H, H), lambda i: (0, 0)),
          pl.BlockSpec((1, H), lambda i: (0, 0)),
          pl.BlockSpec((H, O), lambda i: (0, 0)),
          pl.BlockSpec((1, O), lambda i: (0, 0)),
      ],
      out_specs=pl.BlockSpec((G, O), lambda i: (0, 0)),
      out_shape=jax.ShapeDtypeStruct((G, O), jnp.float32),
      scratch_shapes=[
          pltpu.VMEM((G, D), jnp.float32),
          pltpu.VMEM((G, 1), jnp.float32),
      ],
      compiler_params=pltpu.CompilerParams(
          dimension_semantics=("arbitrary",),
      ),
  )(batch3d, x, W1, b1.reshape(1, H), W2, b2.reshape(1, H), W3,
    b3.reshape(1, O))
  return out


# speedup vs baseline: 5.0535x; 1.6520x over previous
import functools

import jax
import jax.numpy as jnp
from jax import lax
from jax.experimental import pallas as pl
from jax.experimental.pallas import tpu as pltpu
from jax.experimental.pallas import tpu_sc as plsc

N = 100000
D = 128
H = 256
O = 10
G = 256

NC = 2
NS = 16
NW = NC * NS
RPW = N // NW
CH = 125
NCHUNK = RPW // CH

_mesh = plsc.VectorSubcoreMesh(
    core_axis_name="c", subcore_axis_name="s", num_cores=NC, num_subcores=NS)


@functools.partial(
    pl.kernel,
    out_type=[
        jax.ShapeDtypeStruct((NC, G, D), jnp.float32),
        jax.ShapeDtypeStruct((NC, G, 16), jnp.float32),
    ],
    mesh=_mesh,
    scratch_types=[
        pltpu.VMEM((CH, D), jnp.float32),
        pltpu.VMEM((NCHUNK, CH), jnp.int32),
        pltpu.VMEM((CH, 16), jnp.float32),
        pltpu.VMEM_SHARED((G, D), jnp.float32),
        pltpu.VMEM_SHARED((G, 16), jnp.float32),
    ],
    compiler_params=pltpu.CompilerParams(use_tc_tiling_on_sc=False),
)
def _seg_sc(x_hbm, batch_hbm, zsum_hbm, zcnt_hbm, ones_hbm,
            sums_out, cnt_out, chunk_v, idx_v, ones_v, sums_sp, cnt_sp):
  cid = lax.axis_index("c")
  sid = lax.axis_index("s")
  wid = sid * NC + cid
  base = wid * RPW

  pltpu.sync_copy(batch_hbm.at[wid], idx_v)
  pltpu.sync_copy(ones_hbm, ones_v)

  @pl.when(sid == 0)
  def _init():
    pltpu.sync_copy(zsum_hbm, sums_sp)
    pltpu.sync_copy(zcnt_hbm, cnt_sp)

  plsc.subcore_barrier()

  def body(c, carry):
    pltpu.sync_copy(x_hbm.at[pl.ds(base + c * CH, CH)], chunk_v)
    pltpu.sync_copy(chunk_v, sums_sp.at[idx_v.at[c]], add=True)
    pltpu.sync_copy(ones_v, cnt_sp.at[idx_v.at[c]], add=True)
    return carry

  lax.fori_loop(0, NCHUNK, body, 0)

  plsc.subcore_barrier()

  @pl.when(sid == 0)
  def _writeout():
    pltpu.sync_copy(sums_sp, sums_out.at[cid])
    pltpu.sync_copy(cnt_sp, cnt_out.at[cid])


def _mlp_kernel(sums_ref, cnt_ref, w1_ref, b1_ref, w2_ref, b2_ref,
                w3_ref, b3_ref, out_ref):
  sums = sums_ref[0] + sums_ref[1]
  cnt = cnt_ref[0, :, 0:1] + cnt_ref[1, :, 0:1]
  pooled = sums / jnp.maximum(cnt, 1.0)
  h = jnp.maximum(
      jnp.dot(pooled, w1_ref[...], preferred_element_type=jnp.float32)
      + b1_ref[...], 0.0)
  h = jnp.maximum(
      jnp.dot(h, w2_ref[...], preferred_element_type=jnp.float32)
      + b2_ref[...], 0.0)
  out_ref[...] = (
      jnp.dot(h, w3_ref[...], preferred_element_type=jnp.float32)
      + b3_ref[...])


@jax.jit
def kernel(x, batch, W1, b1, W2, b2, W3, b3):
  batch3 = batch.astype(jnp.int32).reshape(NW, NCHUNK, CH)
  zsum = jnp.zeros((G, D), jnp.float32)
  zcnt = jnp.zeros((G, 16), jnp.float32)
  ones = jnp.ones((CH, 16), jnp.float32)
  sums2, cnt2 = _seg_sc(x, batch3, zsum, zcnt, ones)
  out = pl.pallas_call(
      _mlp_kernel,
      out_shape=jax.ShapeDtypeStruct((G, O), jnp.float32),
  )(sums2, cnt2, W1, b1.reshape(1, H), W2, b2.reshape(1, H), W3,
    b3.reshape(1, O))
  return out


# speedup vs baseline: 5.8270x; 1.1531x over previous
import functools

import jax
import jax.numpy as jnp
from jax import lax
from jax.experimental import pallas as pl
from jax.experimental.pallas import tpu as pltpu
from jax.experimental.pallas import tpu_sc as plsc

N = 100000
D = 128
H = 256
O = 10
G = 256

NC = 2
NS = 16
NW = NC * NS
RPW = N // NW
CH = 125
NCHUNK = RPW // CH

_mesh = plsc.VectorSubcoreMesh(
    core_axis_name="c", subcore_axis_name="s", num_cores=NC, num_subcores=NS)


@functools.partial(
    pl.kernel,
    out_type=[
        jax.ShapeDtypeStruct((NC, G, D), jnp.float32),
        jax.ShapeDtypeStruct((NC, G, 16), jnp.float32),
    ],
    mesh=_mesh,
    scratch_types=[
        pltpu.VMEM((CH, D), jnp.float32),
        pltpu.VMEM((CH, D), jnp.float32),
        pltpu.VMEM((NCHUNK, CH), jnp.int32),
        pltpu.VMEM((CH, 16), jnp.float32),
        pltpu.VMEM_SHARED((G, D), jnp.float32),
        pltpu.VMEM_SHARED((G, 16), jnp.float32),
        pltpu.SemaphoreType.DMA,
        pltpu.SemaphoreType.DMA,
        pltpu.SemaphoreType.DMA,
        pltpu.SemaphoreType.DMA,
    ],
    compiler_params=pltpu.CompilerParams(use_tc_tiling_on_sc=False),
)
def _seg_sc(x_hbm, batch_hbm, zsum_hbm, zcnt_hbm, ones_hbm,
            sums_out, cnt_out, buf_a, buf_b, idx_v, ones_v, sums_sp, cnt_sp,
            sem_a, sem_b, sem_s, sem_c):
  cid = lax.axis_index("c")
  sid = lax.axis_index("s")
  wid = sid * NC + cid
  base = wid * RPW

  def start_fetch(c, buf, sem):
    pltpu.async_copy(x_hbm.at[pl.ds(base + c * CH, CH)], buf, sem)

  def wait_fetch(buf, sem):
    pltpu.make_async_copy(x_hbm.at[pl.ds(base, CH)], buf, sem).wait()

  def scatter(c, buf):
    ds = pltpu.async_copy(buf, sums_sp.at[idx_v.at[c]], sem_s, add=True)
    dc = pltpu.async_copy(ones_v, cnt_sp.at[idx_v.at[c]], sem_c, add=True)
    ds.wait()
    dc.wait()

  start_fetch(0, buf_a, sem_a)
  pltpu.sync_copy(batch_hbm.at[wid], idx_v)
  pltpu.sync_copy(ones_hbm, ones_v)

  @pl.when(sid == 0)
  def _init():
    pltpu.sync_copy(zsum_hbm, sums_sp)
    pltpu.sync_copy(zcnt_hbm, cnt_sp)

  plsc.subcore_barrier()

  def body(k, carry):
    c = 2 * k
    start_fetch(c + 1, buf_b, sem_b)
    wait_fetch(buf_a, sem_a)
    scatter(c, buf_a)
    start_fetch(c + 2, buf_a, sem_a)
    wait_fetch(buf_b, sem_b)
    scatter(c + 1, buf_b)
    return carry

  lax.fori_loop(0, (NCHUNK - 1) // 2, body, 0)
  wait_fetch(buf_a, sem_a)
  scatter(NCHUNK - 1, buf_a)

  plsc.subcore_barrier()

  @pl.when(sid == 0)
  def _writeout():
    pltpu.sync_copy(sums_sp, sums_out.at[cid])
    pltpu.sync_copy(cnt_sp, cnt_out.at[cid])


def _mlp_kernel(sums_ref, cnt_ref, w1_ref, b1_ref, w2_ref, b2_ref,
                w3_ref, b3_ref, out_ref):
  sums = sums_ref[0] + sums_ref[1]
  cnt = cnt_ref[0, :, 0:1] + cnt_ref[1, :, 0:1]
  pooled = sums / jnp.maximum(cnt, 1.0)
  h = jnp.maximum(
      jnp.dot(pooled, w1_ref[...], preferred_element_type=jnp.float32)
      + b1_ref[...], 0.0)
  h = jnp.maximum(
      jnp.dot(h, w2_ref[...], preferred_element_type=jnp.float32)
      + b2_ref[...], 0.0)
  out_ref[...] = (
      jnp.dot(h, w3_ref[...], preferred_element_type=jnp.float32)
      + b3_ref[...])


@jax.jit
def kernel(x, batch, W1, b1, W2, b2, W3, b3):
  batch3 = batch.astype(jnp.int32).reshape(NW, NCHUNK, CH)
  zsum = jnp.zeros((G, D), jnp.float32)
  zcnt = jnp.zeros((G, 16), jnp.float32)
  ones = jnp.ones((CH, 16), jnp.float32)
  sums2, cnt2 = _seg_sc(x, batch3, zsum, zcnt, ones)
  out = pl.pallas_call(
      _mlp_kernel,
      out_shape=jax.ShapeDtypeStruct((G, O), jnp.float32),
  )(sums2, cnt2, W1, b1.reshape(1, H), W2, b2.reshape(1, H), W3,
    b3.reshape(1, O))
  return out
